# baseline (speedup 1.0000x reference)
import jax
import jax.numpy as jnp
from jax.experimental import pallas as pl
from jax.experimental.pallas import tpu as pltpu

N = 512
B = 4
HID = 64
EMB = 128
NC = 2


def _bf16_dot(a, b):
    return jnp.dot(a.astype(jnp.bfloat16), b.astype(jnp.bfloat16),
                   preferred_element_type=jnp.float32)


def _bf16_dot_t(a, b):
    return jax.lax.dot_general(
        a.astype(jnp.bfloat16), b.astype(jnp.bfloat16),
        (((1,), (1,)), ((), ())), preferred_element_type=jnp.float32)


def _dot_t0(u, t):
    return jax.lax.dot_general(
        u, t.astype(jnp.bfloat16), (((0,), (0,)), ((), ())),
        preferred_element_type=jnp.float32)


def _encode_one(a, w1, b1, w2, b2, row, col, ones8):
    up16 = jnp.where(col > row, a, 0.0).astype(jnp.bfloat16)
    rs = jnp.dot(up16, ones8, preferred_element_type=jnp.float32)[:, 0:1]
    cs = _dot_t0(up16, ones8)[:, 0:1]
    deg = rs + cs + 1.0
    dinv = jax.lax.rsqrt(deg)

    h0 = _bf16_dot_t(a, w1)
    t1 = h0 * dinv
    s1 = _bf16_dot(up16, t1) + _dot_t0(up16, t1) + t1
    h1 = jnp.maximum(s1 * dinv + b1, 0.0)

    g0 = _bf16_dot_t(h1, w2)
    t2 = g0 * dinv
    s2 = _bf16_dot(up16, t2) + _dot_t0(up16, t2) + t2
    h2 = jnp.maximum(s2 * dinv + b2, 0.0)
    return jnp.mean(h2, axis=0)


def _fused_body(fc_ref, sc_ref, fw1_ref, fb1_ref, fw2_ref, fb2_ref,
                sw1_ref, sb1_ref, sw2_ref, sb2_ref,
                hw1_ref, hb1_ref, hw2_ref, hb2_ref, out_ref, feat_ref):
    b = pl.program_id(0)
    row = jax.lax.broadcasted_iota(jnp.int32, (N, N), 0)
    col = jax.lax.broadcasted_iota(jnp.int32, (N, N), 1)

    ones8 = jnp.ones((N, 8), jnp.bfloat16)
    fc_emb = _encode_one(fc_ref[0], fw1_ref[...], fb1_ref[0], fw2_ref[...],
                         fb2_ref[0], row, col, ones8)
    sc_emb = _encode_one(sc_ref[0], sw1_ref[...], sb1_ref[0], sw2_ref[...],
                         sb2_ref[0], row, col, ones8)
    feat_ref[pl.ds(b, 1), :] = jnp.concatenate([fc_emb, sc_emb])[None, :]

    @pl.when(b == B - 1)
    def _():
        feat = feat_ref[...]
        h = jnp.maximum(_bf16_dot_t(feat, hw1_ref[...]) + hb1_ref[0], 0.0)
        out_ref[...] = _bf16_dot_t(h, hw2_ref[...]) + hb2_ref[0]


def kernel(fc_adj, sc_adj, fc_W1, fc_b1, fc_W2, fc_b2,
           sc_W1, sc_b1, sc_W2, sc_b2, head_W1, head_b1, head_W2, head_b2):
    full = lambda shape: pl.BlockSpec(shape, lambda b: tuple(0 for _ in shape))
    return pl.pallas_call(
        _fused_body,
        grid=(B,),
        in_specs=[
            pl.BlockSpec((1, N, N), lambda b: (b, 0, 0)),
            pl.BlockSpec((1, N, N), lambda b: (b, 0, 0)),
            full((HID, N)), full((1, HID)), full((EMB, HID)), full((1, EMB)),
            full((HID, N)), full((1, HID)), full((EMB, HID)), full((1, EMB)),
            full((2 * HID, 2 * EMB)), full((1, 2 * HID)),
            full((NC, 2 * HID)), full((1, NC)),
        ],
        out_specs=pl.BlockSpec((B, NC), lambda b: (0, 0)),
        out_shape=jax.ShapeDtypeStruct((B, NC), jnp.float32),
        scratch_shapes=[pltpu.VMEM((B, 2 * EMB), jnp.float32)],
    )(fc_adj, sc_adj,
      fc_W1, fc_b1.reshape(1, HID), fc_W2, fc_b2.reshape(1, EMB),
      sc_W1, sc_b1.reshape(1, HID), sc_W2, sc_b2.reshape(1, EMB),
      head_W1, head_b1.reshape(1, 2 * HID), head_W2, head_b2.reshape(1, NC))


# speedup vs baseline: 1.3596x; 1.3596x over previous
import jax
import jax.numpy as jnp
from jax.experimental import pallas as pl
from jax.experimental.pallas import tpu as pltpu

N = 512
B = 4
HID = 64
EMB = 128
NC = 2


def _bf16_dot(a, b):
    return jnp.dot(a.astype(jnp.bfloat16), b.astype(jnp.bfloat16),
                   preferred_element_type=jnp.float32)


def _bf16_dot_t(a, b):
    return jax.lax.dot_general(
        a.astype(jnp.bfloat16), b.astype(jnp.bfloat16),
        (((1,), (1,)), ((), ())), preferred_element_type=jnp.float32)


def _encode_one(a, w1, b1, w2, b2, row, col):
    a16 = a.astype(jnp.bfloat16)
    up16 = jnp.where(col > row, a16, jnp.bfloat16(0.0))
    abar16 = up16 + up16.T
    deg = jnp.sum(abar16, axis=1, dtype=jnp.float32) + 1.0
    dinv = jax.lax.rsqrt(deg)[:, None]

    h0 = _bf16_dot_t(a16, w1)
    t1 = h0 * dinv
    s1 = _bf16_dot(abar16, t1) + t1
    h1 = jnp.maximum(s1 * dinv + b1, 0.0)

    g0 = _bf16_dot_t(h1, w2)
    t2 = g0 * dinv
    s2 = _bf16_dot(abar16, t2) + t2
    h2 = jnp.maximum(s2 * dinv + b2, 0.0)
    return jnp.mean(h2, axis=0)


def _fused_body(fc_ref, sc_ref, fw1_ref, fb1_ref, fw2_ref, fb2_ref,
                sw1_ref, sb1_ref, sw2_ref, sb2_ref,
                hw1_ref, hb1_ref, hw2_ref, hb2_ref, out_ref, feat_ref):
    b = pl.program_id(0)
    row = jax.lax.broadcasted_iota(jnp.int32, (N, N), 0)
    col = jax.lax.broadcasted_iota(jnp.int32, (N, N), 1)

    fc_emb = _encode_one(fc_ref[0], fw1_ref[...], fb1_ref[0], fw2_ref[...],
                         fb2_ref[0], row, col)
    sc_emb = _encode_one(sc_ref[0], sw1_ref[...], sb1_ref[0], sw2_ref[...],
                         sb2_ref[0], row, col)
    feat_ref[pl.ds(b, 1), :] = jnp.concatenate([fc_emb, sc_emb])[None, :]

    @pl.when(b == B - 1)
    def _():
        feat = feat_ref[...]
        h = jnp.maximum(_bf16_dot_t(feat, hw1_ref[...]) + hb1_ref[0], 0.0)
        out_ref[...] = _bf16_dot_t(h, hw2_ref[...]) + hb2_ref[0]


def kernel(fc_adj, sc_adj, fc_W1, fc_b1, fc_W2, fc_b2,
           sc_W1, sc_b1, sc_W2, sc_b2, head_W1, head_b1, head_W2, head_b2):
    full = lambda shape: pl.BlockSpec(shape, lambda b: tuple(0 for _ in shape))
    return pl.pallas_call(
        _fused_body,
        grid=(B,),
        in_specs=[
            pl.BlockSpec((1, N, N), lambda b: (b, 0, 0)),
            pl.BlockSpec((1, N, N), lambda b: (b, 0, 0)),
            full((HID, N)), full((1, HID)), full((EMB, HID)), full((1, EMB)),
            full((HID, N)), full((1, HID)), full((EMB, HID)), full((1, EMB)),
            full((2 * HID, 2 * EMB)), full((1, 2 * HID)),
            full((NC, 2 * HID)), full((1, NC)),
        ],
        out_specs=pl.BlockSpec((B, NC), lambda b: (0, 0)),
        out_shape=jax.ShapeDtypeStruct((B, NC), jnp.float32),
        scratch_shapes=[pltpu.VMEM((B, 2 * EMB), jnp.float32)],
    )(fc_adj, sc_adj,
      fc_W1, fc_b1.reshape(1, HID), fc_W2, fc_b2.reshape(1, EMB),
      sc_W1, sc_b1.reshape(1, HID), sc_W2, sc_b2.reshape(1, EMB),
      head_W1, head_b1.reshape(1, 2 * HID), head_W2, head_b2.reshape(1, NC))
